# Initial kernel scaffold; baseline (speedup 1.0000x reference)
import jax
import jax.numpy as jnp
from jax.experimental import pallas as pl


def kernel(x, edge_index, edge_attr, rev_edge_index, W_i, b_i, W_h, b_h, W_o, b_o):
    raise NotImplementedError("write your pallas kernel here")


# speedup vs baseline: 1.4462x; 1.4462x over previous
import functools

import jax
import jax.numpy as jnp
from jax.experimental import pallas as pl
from jax.experimental.pallas import tpu as pltpu

_DEPTH = 5
_R = 2000


def _k1_body(xg_ref, ea_ref, w1_ref, w2_ref, b_ref, o_ref):
    acc = jnp.dot(xg_ref[...], w1_ref[...], preferred_element_type=jnp.float32)
    acc += jnp.dot(ea_ref[...], w2_ref[...], preferred_element_type=jnp.float32)
    o_ref[...] = jnp.maximum(acc + b_ref[...], 0.0)


def _k1(xg, ea, w1, w2, b):
    e, d = xg.shape
    bd = ea.shape[1]
    h = w1.shape[1]
    return pl.pallas_call(
        _k1_body,
        grid=(e // _R,),
        in_specs=[
            pl.BlockSpec((_R, d), lambda i: (i, 0)),
            pl.BlockSpec((_R, bd), lambda i: (i, 0)),
            pl.BlockSpec((d, h), lambda i: (0, 0)),
            pl.BlockSpec((bd, h), lambda i: (0, 0)),
            pl.BlockSpec((1, h), lambda i: (0, 0)),
        ],
        out_specs=pl.BlockSpec((_R, h), lambda i: (i, 0)),
        out_shape=jax.ShapeDtypeStruct((e, h), jnp.float32),
    )(xg, ea, w1, w2, b)


def _k2_body(m_ref, h0_ref, w_ref, b_ref, o_ref):
    acc = jnp.dot(m_ref[...], w_ref[...], preferred_element_type=jnp.float32)
    o_ref[...] = jnp.maximum(h0_ref[...] + acc + b_ref[...], 0.0)


def _k2(m, h0, w, b):
    e, h = m.shape
    return pl.pallas_call(
        _k2_body,
        grid=(e // _R,),
        in_specs=[
            pl.BlockSpec((_R, h), lambda i: (i, 0)),
            pl.BlockSpec((_R, h), lambda i: (i, 0)),
            pl.BlockSpec((h, h), lambda i: (0, 0)),
            pl.BlockSpec((1, h), lambda i: (0, 0)),
        ],
        out_specs=pl.BlockSpec((_R, h), lambda i: (i, 0)),
        out_shape=jax.ShapeDtypeStruct((e, h), jnp.float32),
    )(m, h0, w, b)


def kernel(x, edge_index, edge_attr, rev_edge_index, W_i, b_i, W_h, b_h, W_o, b_o):
    n, d = x.shape
    ei0 = edge_index[0]
    ei1 = edge_index[1]

    b_i2 = b_i.reshape(1, -1)
    b_h2 = b_h.reshape(1, -1)
    b_o2 = b_o.reshape(1, -1)

    xg = x[ei0]
    h0 = _k1(xg, edge_attr, W_i[:d], W_i[d:], b_i2)
    h = h0
    for _ in range(1, _DEPTH):
        m = jnp.zeros_like(h).at[ei1].add(h)
        m = m.at[ei0].add(-h[rev_edge_index])
        h = _k2(m, h0, W_h, b_h2)
    m_final = jnp.zeros_like(h).at[ei1].add(h)
    out = _k1(x, m_final, W_o[:d], W_o[d:], b_o2)
    return out
